# baseline (speedup 1.0000x reference)
import jax
import jax.numpy as jnp
from jax.experimental import pallas as pl
from jax.experimental.pallas import tpu as pltpu

THRESHOLD = 0.05
WEIGHT = 1.0

B = 8
N = 16384
M = 2048
BN = 16384
NBLK = N // BN
CH = 256
NCH = BN // CH


def _loss_kernel(pg_ref, at_ref, c2l_ref, out_ref, s_ref, n_ref, p2s_ref):
    b = pl.program_id(0)
    i = pl.program_id(1)
    step = b * NBLK + i

    @pl.when(step == 0)
    def _init():
        s_ref[...] = jnp.zeros_like(s_ref)
        n_ref[...] = jnp.zeros_like(n_ref)

    pg = pg_ref[0]
    at = at_ref[0]
    c2l = c2l_ref[0]

    @pl.when(i == 0)
    def _batch_setup():
        p2 = jnp.sum(pg * pg, axis=1, keepdims=True)
        p2s_ref[...] = jnp.broadcast_to(p2, (M, CH))

    p2s = p2s_ref[...]

    svec = jnp.zeros((1, CH), jnp.float32)
    nvec = jnp.zeros((1, CH), jnp.float32)
    for q in range(NCH):
        atc = at[:, q * CH:(q + 1) * CH]
        e = jnp.dot(pg, atc, preferred_element_type=jnp.float32)
        e = e + p2s
        rows = M
        while rows > 8:
            half = rows // 2
            e = jnp.minimum(e[:half], e[half:rows])
            rows = half
        dminc = jnp.min(e, axis=0, keepdims=True)
        dminc = jnp.maximum(dminc + c2l[q:q + 1, :], 0.0)
        mask = dminc < THRESHOLD
        svec = svec + jnp.where(mask, dminc, 0.0)
        nvec = nvec + mask.astype(jnp.float32)

    s_ref[...] += svec
    n_ref[...] += nvec

    @pl.when(step == B * NBLK - 1)
    def _finish():
        s = jnp.sum(s_ref[...])
        mm = jnp.sum(n_ref[...])
        out_ref[0, 0] = jnp.where(mm > 0.0, s / (mm + 1e-6), 0.0)


@jax.jit
def kernel(completed, partial):
    ci = jnp.argsort(completed[..., 0], axis=1)
    completed = jnp.take_along_axis(completed, ci[..., None], axis=1)
    pi = jnp.argsort(partial[..., 0], axis=1)
    partial = jnp.take_along_axis(partial, pi[..., None], axis=1)
    pg = jnp.pad(partial, ((0, 0), (0, 0), (0, 5)))
    at = jnp.transpose(-2.0 * completed, (0, 2, 1))
    at = jnp.pad(at, ((0, 0), (0, 5), (0, 0)))
    c2 = jnp.sum(completed * completed, axis=-1)
    c2l = c2.reshape(B, N // CH, CH)

    out = pl.pallas_call(
        _loss_kernel,
        grid=(B, NBLK),
        in_specs=[
            pl.BlockSpec((1, M, 8), lambda b, i: (b, 0, 0)),
            pl.BlockSpec((1, 8, BN), lambda b, i: (b, 0, i)),
            pl.BlockSpec((1, BN // CH, CH), lambda b, i: (b, i, 0)),
        ],
        out_specs=pl.BlockSpec(memory_space=pltpu.SMEM),
        out_shape=jax.ShapeDtypeStruct((1, 1), jnp.float32),
        scratch_shapes=[
            pltpu.VMEM((1, CH), jnp.float32),
            pltpu.VMEM((1, CH), jnp.float32),
            pltpu.VMEM((M, CH), jnp.float32),
        ],
    )(pg, at, c2l)
    return WEIGHT * out[0, 0]


# speedup vs baseline: 2.5450x; 2.5450x over previous
import jax
import jax.numpy as jnp
from jax.experimental import pallas as pl
from jax.experimental.pallas import tpu as pltpu

THRESHOLD = 0.05
WEIGHT = 1.0

B = 8
N = 16384
M = 2048
BN = 16384
NBLK = N // BN
CH = 256
NCH = BN // CH


def _loss_kernel(pg_ref, at_ref, out_ref, s_ref, n_ref, p2s_ref):
    b = pl.program_id(0)
    i = pl.program_id(1)
    step = b * NBLK + i

    @pl.when(step == 0)
    def _init():
        s_ref[...] = jnp.zeros_like(s_ref)
        n_ref[...] = jnp.zeros_like(n_ref)

    pg = pg_ref[0]
    at = at_ref[0]

    @pl.when(i == 0)
    def _batch_setup():
        p2 = jnp.sum(pg * pg, axis=1, keepdims=True)
        p2s_ref[...] = jnp.broadcast_to(p2, (M, CH))

    p2s = p2s_ref[...]

    svec = jnp.zeros((1, CH), jnp.float32)
    nvec = jnp.zeros((1, CH), jnp.float32)
    for q in range(NCH):
        atc = at[:, q * CH:(q + 1) * CH]
        e = jnp.dot(pg, atc, preferred_element_type=jnp.float32)
        e = e + p2s
        rows = M
        while rows > 8:
            half = rows // 2
            e = jnp.minimum(e[:half], e[half:rows])
            rows = half
        dminc = jnp.min(e, axis=0, keepdims=True)
        c2row = jnp.sum(atc * atc, axis=0, keepdims=True) * 0.25
        dminc = jnp.maximum(dminc + c2row, 0.0)
        mask = dminc < THRESHOLD
        svec = svec + jnp.where(mask, dminc, 0.0)
        nvec = nvec + mask.astype(jnp.float32)

    s_ref[...] += svec
    n_ref[...] += nvec

    @pl.when(step == B * NBLK - 1)
    def _finish():
        s = jnp.sum(s_ref[...])
        mm = jnp.sum(n_ref[...])
        out_ref[0, 0] = jnp.where(mm > 0.0, s / (mm + 1e-6), 0.0)


@jax.jit
def kernel(completed, partial):
    pg = jnp.pad(partial, ((0, 0), (0, 0), (0, 5)))
    at = jnp.transpose(-2.0 * completed, (0, 2, 1))
    at = jnp.pad(at, ((0, 0), (0, 5), (0, 0)))

    out = pl.pallas_call(
        _loss_kernel,
        grid=(B, NBLK),
        in_specs=[
            pl.BlockSpec((1, M, 8), lambda b, i: (b, 0, 0)),
            pl.BlockSpec((1, 8, BN), lambda b, i: (b, 0, i)),
        ],
        out_specs=pl.BlockSpec(memory_space=pltpu.SMEM),
        out_shape=jax.ShapeDtypeStruct((1, 1), jnp.float32),
        scratch_shapes=[
            pltpu.VMEM((1, CH), jnp.float32),
            pltpu.VMEM((1, CH), jnp.float32),
            pltpu.VMEM((M, CH), jnp.float32),
        ],
    )(pg, at)
    return WEIGHT * out[0, 0]
